# baseline (speedup 1.0000x reference)
import jax
import jax.numpy as jnp
from jax.experimental import pallas as pl
from jax.experimental.pallas import tpu as pltpu

_BLOCK_ROWS = 1024


def _binarize_pack_kernel(x_ref, m_ref, o_ref):
    m = m_ref[...]
    c = jnp.logical_and(m > 0.0, x_ref[...] >= m)
    f8 = o_ref.shape[-1]
    acc = None
    for k in range(8):
        piece = c[:, k * f8:(k + 1) * f8]
        v = jnp.where(piece, jnp.int32(1 << k), jnp.int32(0))
        acc = v if acc is None else acc | v
    o_ref[...] = acc.astype(jnp.uint8)


def kernel(inputs, medians):
    n, f = inputs.shape
    f8 = f // 8
    m2 = medians.reshape(1, f)
    grid = (n // _BLOCK_ROWS,)
    packed = pl.pallas_call(
        _binarize_pack_kernel,
        grid=grid,
        in_specs=[
            pl.BlockSpec((_BLOCK_ROWS, f), lambda i: (i, 0)),
            pl.BlockSpec((1, f), lambda i: (0, 0)),
        ],
        out_specs=pl.BlockSpec((_BLOCK_ROWS, f8), lambda i: (i, 0)),
        out_shape=jax.ShapeDtypeStruct((n, f8), jnp.uint8),
        compiler_params=pltpu.CompilerParams(
            dimension_semantics=("parallel",),
        ),
    )(inputs, m2)
    bits = jnp.arange(8, dtype=jnp.uint8)
    out = (packed[:, None, :] >> bits[None, :, None]) & jnp.uint8(1)
    return out.astype(jnp.bool_).reshape(n, f)


# speedup vs baseline: 1.1433x; 1.1433x over previous
import jax
import jax.numpy as jnp
from jax.experimental import pallas as pl
from jax.experimental.pallas import tpu as pltpu

_BLOCK_ROWS = 1024


def _binarize_pack_kernel(x_ref, m_ref, o_ref):
    m = m_ref[...]
    c = jnp.logical_and(m > 0.0, x_ref[...] >= m)
    f8 = o_ref.shape[-1]
    acc = None
    for k in range(8):
        piece = c[:, k * f8:(k + 1) * f8]
        v = jnp.where(piece, jnp.int32(1 << k), jnp.int32(0))
        acc = v if acc is None else acc | v
    o_ref[...] = acc.astype(jnp.uint8)


def kernel(inputs, medians):
    n, f = inputs.shape
    f8 = f // 8
    m2 = medians.reshape(1, f)
    grid = (n // _BLOCK_ROWS,)
    packed = pl.pallas_call(
        _binarize_pack_kernel,
        grid=grid,
        in_specs=[
            pl.BlockSpec((_BLOCK_ROWS, f), lambda i: (i, 0)),
            pl.BlockSpec((1, f), lambda i: (0, 0)),
        ],
        out_specs=pl.BlockSpec((_BLOCK_ROWS, f8), lambda i: (i, 0)),
        out_shape=jax.ShapeDtypeStruct((n, f8), jnp.uint8),
        compiler_params=pltpu.CompilerParams(
            dimension_semantics=("parallel",),
        ),
    )(inputs, m2)
    return jnp.concatenate(
        [(packed & jnp.uint8(1 << k)) != 0 for k in range(8)], axis=1
    )


# speedup vs baseline: 1.4875x; 1.3011x over previous
import jax
import jax.numpy as jnp
from jax import lax
from jax.experimental import pallas as pl
from jax.experimental.pallas import tpu as pltpu

_BLOCK_ROWS = 1024


def _binarize_pack_kernel(x_ref, m_ref, o_ref):
    m = m_ref[...]
    c = jnp.logical_and(m > 0.0, x_ref[...] >= m)
    br, f = x_ref.shape
    row_bit = lax.broadcasted_iota(jnp.int32, (br, f), 0) % 8
    w = jnp.where(c, jnp.int32(1) << row_bit, jnp.int32(0))
    packed = jnp.sum(w.reshape(br // 8, 8, f), axis=1)
    o_ref[...] = packed.astype(jnp.uint8)


def kernel(inputs, medians):
    n, f = inputs.shape
    m2 = medians.reshape(1, f)
    grid = (n // _BLOCK_ROWS,)
    packed = pl.pallas_call(
        _binarize_pack_kernel,
        grid=grid,
        in_specs=[
            pl.BlockSpec((_BLOCK_ROWS, f), lambda i: (i, 0)),
            pl.BlockSpec((1, f), lambda i: (0, 0)),
        ],
        out_specs=pl.BlockSpec((_BLOCK_ROWS // 8, f), lambda i: (i, 0)),
        out_shape=jax.ShapeDtypeStruct((n // 8, f), jnp.uint8),
        compiler_params=pltpu.CompilerParams(
            dimension_semantics=("parallel",),
        ),
    )(inputs, m2)
    bits = jnp.arange(8, dtype=jnp.uint8)
    out3 = (packed[:, None, :] >> bits[None, :, None]) & jnp.uint8(1)
    return (out3 != 0).reshape(n, f)
